# baseline (speedup 1.0000x reference)
import functools

import jax
import jax.numpy as jnp
from jax import lax
from jax.experimental import pallas as pl
from jax.experimental.pallas import tpu as pltpu
from jax.experimental.pallas import tpu_sc as plsc

N = 10000
D = 256
H = 128
E = 160000
SCALE = 1.8

NC = 2
NS = 16
W = 128
CHW = 40

NPAD = 10496
STRIPE = NPAD // NS

EPAD = 163840
ED_W = EPAD // (NC * NS * W)
EP_W = EPAD // (NS * W)

_mesh = plsc.VectorSubcoreMesh(core_axis_name="c", subcore_axis_name="s")


@functools.partial(
    pl.kernel,
    out_type=jax.ShapeDtypeStruct((NC * NPAD, H), jnp.float32),
    mesh=_mesh,
    scratch_types=[
        pltpu.VMEM((ED_W, W), jnp.int32),
        pltpu.VMEM((W, H), jnp.float32),
        pltpu.VMEM_SHARED((NPAD, H), jnp.float32),
    ],
)
def _sc_deg(dst_hbm, ones_hbm, z_hbm, out_hbm, idx_v, ones_v, acc_s):
    c = lax.axis_index("c")
    s = lax.axis_index("s")
    pltpu.sync_copy(ones_hbm, ones_v)
    pltpu.sync_copy(dst_hbm.at[c * NS + s], idx_v)
    pltpu.sync_copy(z_hbm.at[pl.ds(s * STRIPE, STRIPE)],
                    acc_s.at[pl.ds(s * STRIPE, STRIPE)])
    plsc.subcore_barrier()

    @pl.loop(0, ED_W)
    def _(k):
        pltpu.sync_copy(ones_v, acc_s.at[idx_v.at[k]], add=True)

    plsc.subcore_barrier()
    pltpu.sync_copy(acc_s.at[pl.ds(s * STRIPE, STRIPE)],
                    out_hbm.at[pl.ds(c * NPAD + s * STRIPE, STRIPE)])


@functools.partial(
    pl.kernel,
    out_type=[jax.ShapeDtypeStruct((NPAD, H), jnp.float32)] * 4,
    mesh=_mesh,
    scratch_types=[
        pltpu.VMEM((CHW, W), jnp.int32),
        pltpu.VMEM((CHW, W), jnp.int32),
        pltpu.VMEM((W, H), jnp.float32),
        pltpu.VMEM((W, H), jnp.float32),
        pltpu.VMEM_SHARED((NPAD, H), jnp.float32),
        pltpu.SemaphoreType.DMA,
        pltpu.SemaphoreType.DMA,
        pltpu.SemaphoreType.DMA,
        pltpu.SemaphoreType.DMA,
    ],
)
def _sc_prop(src_hbm, dst_hbm, t0, t1, t2, t3, z_hbm, o0, o1, o2, o3,
             idxs_v, idxd_v, upd_a, upd_b, acc_s,
             sem_ga, sem_gb, sem_sa, sem_sb):
    c = lax.axis_index("c")
    s = lax.axis_index("s")

    def one_pass(table, out):
        pltpu.sync_copy(z_hbm.at[pl.ds(s * STRIPE, STRIPE)],
                        acc_s.at[pl.ds(s * STRIPE, STRIPE)])
        plsc.subcore_barrier()

        for ch in range(EP_W // CHW):
            pltpu.sync_copy(src_hbm.at[s, pl.ds(ch * CHW, CHW)], idxs_v)
            pltpu.sync_copy(dst_hbm.at[s, pl.ds(ch * CHW, CHW)], idxd_v)

            pltpu.async_copy(table.at[idxs_v.at[0]], upd_a, sem_ga)

            @pl.loop(0, CHW // 2)
            def _(j):
                k = 2 * j
                pltpu.make_async_copy(
                    table.at[idxs_v.at[k]], upd_a, sem_ga).wait()
                pltpu.async_copy(upd_a, acc_s.at[idxd_v.at[k]], sem_sa,
                                 add=True)

                @pl.when(j > 0)
                def _():
                    pltpu.make_async_copy(
                        upd_b, acc_s.at[idxd_v.at[k - 1]], sem_sb).wait()

                pltpu.async_copy(table.at[idxs_v.at[k + 1]], upd_b, sem_gb)
                pltpu.make_async_copy(
                    table.at[idxs_v.at[k + 1]], upd_b, sem_gb).wait()
                pltpu.async_copy(upd_b, acc_s.at[idxd_v.at[k + 1]], sem_sb,
                                 add=True)
                pltpu.make_async_copy(
                    upd_a, acc_s.at[idxd_v.at[k]], sem_sa).wait()

                @pl.when(k + 2 < CHW)
                def _():
                    pltpu.async_copy(
                        table.at[idxs_v.at[k + 2]], upd_a, sem_ga)

            pltpu.make_async_copy(
                upd_b, acc_s.at[idxd_v.at[CHW - 1]], sem_sb).wait()

        plsc.subcore_barrier()
        pltpu.sync_copy(acc_s.at[pl.ds(s * STRIPE, STRIPE)],
                        out.at[pl.ds(s * STRIPE, STRIPE)])
        plsc.subcore_barrier()

    @pl.when(c == 0)
    def _():
        one_pass(t0, o0)
        one_pass(t1, o1)

    @pl.when(c == 1)
    def _():
        one_pass(t2, o2)
        one_pass(t3, o3)


def _dinv_block(degp_ref):
    deg = 1.0 + degp_ref[0, :, 0:1] + degp_ref[1, :, 0:1]
    return lax.rsqrt(deg)


def _tc_mats_body(x_ref, w1_ref, b1_ref, w2_ref, b2_ref,
                  y1a_ref, y1b_ref, y2a_ref, y2b_ref):
    dn = (((1,), (1,)), ((), ()))
    y1 = lax.dot_general(x_ref[...], w1_ref[...], dn,
                         preferred_element_type=jnp.float32) + b1_ref[...]
    y2 = lax.dot_general(x_ref[...], w2_ref[...], dn,
                         preferred_element_type=jnp.float32) + b2_ref[...]
    nrm = jnp.sqrt(jnp.sum(y2 * y2, axis=1, keepdims=True))
    y2 = SCALE * y2 / jnp.maximum(nrm, 1e-12)
    y1a_ref[...] = y1[:, :H]
    y1b_ref[...] = y1[:, H:]
    y2a_ref[...] = y2[:, :H]
    y2b_ref[...] = y2[:, H:]


def _tc_scale_body(degp_ref, y1a_ref, y1b_ref, y2a_ref, y2b_ref,
                   u1a_ref, u1b_ref, u2a_ref, u2b_ref):
    dinv = _dinv_block(degp_ref)
    u1a_ref[...] = dinv * y1a_ref[...]
    u1b_ref[...] = dinv * y1b_ref[...]
    u2a_ref[...] = dinv * y2a_ref[...]
    u2b_ref[...] = dinv * y2b_ref[...]


def _tc_fin_body(degp_ref, u1a_ref, u1b_ref, u2a_ref, u2b_ref,
                 a1a_ref, a1b_ref, a2a_ref, a2b_ref, h_ref, x__ref):
    dinv = _dinv_block(degp_ref)
    x__ref[...] = dinv * jnp.concatenate(
        [a1a_ref[...] + u1a_ref[...], a1b_ref[...] + u1b_ref[...]], axis=1)
    h_ref[...] = dinv * jnp.concatenate(
        [a2a_ref[...] + u2a_ref[...], a2b_ref[...] + u2b_ref[...]], axis=1)


RB = 400
TC_GRID = N // RB


def _row_spec(w):
    return pl.BlockSpec((RB, w), lambda i: (i, 0))


def _full_spec(h, w):
    return pl.BlockSpec((h, w), lambda i: (0, 0))


_degp_spec = pl.BlockSpec((2, RB, H), lambda i: (0, i, 0))

_tc_mats = pl.pallas_call(
    _tc_mats_body,
    grid=(TC_GRID,),
    in_specs=[_row_spec(D), _full_spec(D, D), _full_spec(1, D),
              _full_spec(D, D), _full_spec(1, D)],
    out_specs=[_row_spec(H)] * 4,
    out_shape=[jax.ShapeDtypeStruct((NPAD, H), jnp.float32)] * 4,
)

_tc_scale = pl.pallas_call(
    _tc_scale_body,
    grid=(TC_GRID,),
    in_specs=[_degp_spec] + [_row_spec(H)] * 4,
    out_specs=[_row_spec(H)] * 4,
    out_shape=[jax.ShapeDtypeStruct((NPAD, H), jnp.float32)] * 4,
)

_tc_fin = pl.pallas_call(
    _tc_fin_body,
    grid=(TC_GRID,),
    in_specs=[_degp_spec] + [_row_spec(H)] * 8,
    out_specs=[_row_spec(D)] * 2,
    out_shape=[jax.ShapeDtypeStruct((N, D), jnp.float32)] * 2,
)


def kernel(x, edge_index, W1, b1, W2, b2):
    src = edge_index[0].astype(jnp.int32)
    dst = edge_index[1].astype(jnp.int32)
    pad = N + (jnp.arange(EPAD - E, dtype=jnp.int32) % (NPAD - N))
    src = jnp.concatenate([src, pad]).reshape(NS, EP_W, W)
    dst = jnp.concatenate([dst, pad]).reshape(NS, EP_W, W)
    dst_deg = dst.reshape(NC * NS, ED_W, W)

    ones = jnp.ones((W, H), jnp.float32)
    z128 = jnp.zeros((NPAD, H), jnp.float32)

    degp = _sc_deg(dst_deg, ones, z128).reshape(2, NPAD, H)
    ys = _tc_mats(x, W1, b1.reshape(1, D), W2, b2.reshape(1, D))
    us = _tc_scale(degp, *ys)
    accs = _sc_prop(src, dst, *us, z128)
    h, x_ = _tc_fin(degp, *us, *accs)
    return (h, x_)


# speedup vs baseline: 1.0103x; 1.0103x over previous
import functools

import jax
import jax.numpy as jnp
from jax import lax
from jax.experimental import pallas as pl
from jax.experimental.pallas import tpu as pltpu
from jax.experimental.pallas import tpu_sc as plsc

N = 10000
D = 256
H = 128
E = 160000
SCALE = 1.8

NC = 2
NS = 16
W = 128
CHW = 40

NPAD = 10496
STRIPE = NPAD // NS

EPAD = 163840
ED_W = EPAD // (NC * NS * W)
EP_W = EPAD // (NS * W)

_mesh = plsc.VectorSubcoreMesh(core_axis_name="c", subcore_axis_name="s")


@functools.partial(
    pl.kernel,
    out_type=jax.ShapeDtypeStruct((NC * NPAD, H), jnp.float32),
    mesh=_mesh,
    scratch_types=[
        pltpu.VMEM((ED_W, W), jnp.int32),
        pltpu.VMEM((W, H), jnp.float32),
        pltpu.VMEM_SHARED((NPAD, H), jnp.float32),
    ],
)
def _sc_deg(dst_hbm, ones_hbm, z_hbm, out_hbm, idx_v, ones_v, acc_s):
    c = lax.axis_index("c")
    s = lax.axis_index("s")
    pltpu.sync_copy(ones_hbm, ones_v)
    pltpu.sync_copy(dst_hbm.at[c * NS + s], idx_v)
    pltpu.sync_copy(z_hbm.at[pl.ds(s * STRIPE, STRIPE)],
                    acc_s.at[pl.ds(s * STRIPE, STRIPE)])
    plsc.subcore_barrier()

    @pl.loop(0, ED_W)
    def _(k):
        pltpu.sync_copy(ones_v, acc_s.at[idx_v.at[k]], add=True)

    plsc.subcore_barrier()
    pltpu.sync_copy(acc_s.at[pl.ds(s * STRIPE, STRIPE)],
                    out_hbm.at[pl.ds(c * NPAD + s * STRIPE, STRIPE)])


@functools.partial(
    pl.kernel,
    out_type=[jax.ShapeDtypeStruct((NPAD, H), jnp.float32)] * 2,
    mesh=_mesh,
    scratch_types=[
        pltpu.VMEM((CHW, W), jnp.int32),
        pltpu.VMEM((CHW, W), jnp.int32),
        pltpu.VMEM((W, H), jnp.float32),
        pltpu.VMEM((W, H), jnp.float32),
        pltpu.VMEM_SHARED((NPAD, H), jnp.float32),
        pltpu.SemaphoreType.DMA,
        pltpu.SemaphoreType.DMA,
        pltpu.SemaphoreType.DMA,
        pltpu.SemaphoreType.DMA,
    ],
)
def _sc_prop(src_hbm, dst_hbm, t0, t1, z_hbm, o0, o1,
             idxs_v, idxd_v, upd_a, upd_b, acc_s,
             sem_ga, sem_gb, sem_sa, sem_sb):
    c = lax.axis_index("c")
    s = lax.axis_index("s")

    def one_pass(table, out):
        pltpu.sync_copy(z_hbm.at[pl.ds(s * STRIPE, STRIPE)],
                        acc_s.at[pl.ds(s * STRIPE, STRIPE)])
        plsc.subcore_barrier()

        for ch in range(EP_W // CHW):
            pltpu.sync_copy(src_hbm.at[s, pl.ds(ch * CHW, CHW)], idxs_v)
            pltpu.sync_copy(dst_hbm.at[s, pl.ds(ch * CHW, CHW)], idxd_v)

            pltpu.async_copy(table.at[idxs_v.at[0]], upd_a, sem_ga)

            @pl.loop(0, CHW // 2)
            def _(j):
                k = 2 * j
                pltpu.make_async_copy(
                    table.at[idxs_v.at[k]], upd_a, sem_ga).wait()
                pltpu.async_copy(upd_a, acc_s.at[idxd_v.at[k]], sem_sa,
                                 add=True)

                @pl.when(j > 0)
                def _():
                    pltpu.make_async_copy(
                        upd_b, acc_s.at[idxd_v.at[k - 1]], sem_sb).wait()

                pltpu.async_copy(table.at[idxs_v.at[k + 1]], upd_b, sem_gb)
                pltpu.make_async_copy(
                    table.at[idxs_v.at[k + 1]], upd_b, sem_gb).wait()
                pltpu.async_copy(upd_b, acc_s.at[idxd_v.at[k + 1]], sem_sb,
                                 add=True)
                pltpu.make_async_copy(
                    upd_a, acc_s.at[idxd_v.at[k]], sem_sa).wait()

                @pl.when(k + 2 < CHW)
                def _():
                    pltpu.async_copy(
                        table.at[idxs_v.at[k + 2]], upd_a, sem_ga)

            pltpu.make_async_copy(
                upd_b, acc_s.at[idxd_v.at[CHW - 1]], sem_sb).wait()

        plsc.subcore_barrier()
        pltpu.sync_copy(acc_s.at[pl.ds(s * STRIPE, STRIPE)],
                        out.at[pl.ds(s * STRIPE, STRIPE)])
        plsc.subcore_barrier()

    @pl.when(c == 0)
    def _():
        one_pass(t0, o0)

    @pl.when(c == 1)
    def _():
        one_pass(t1, o1)


def _dinv_block(degp_ref):
    deg = 1.0 + degp_ref[0, :, 0:1] + degp_ref[1, :, 0:1]
    return lax.rsqrt(deg)


def _tc_mats_body(x_ref, w1_ref, b1_ref, w2_ref, b2_ref,
                  y1a_ref, y1b_ref, y2a_ref, y2b_ref):
    dn = (((1,), (1,)), ((), ()))
    y1 = lax.dot_general(x_ref[...], w1_ref[...], dn,
                         preferred_element_type=jnp.float32) + b1_ref[...]
    y2 = lax.dot_general(x_ref[...], w2_ref[...], dn,
                         preferred_element_type=jnp.float32) + b2_ref[...]
    nrm = jnp.sqrt(jnp.sum(y2 * y2, axis=1, keepdims=True))
    y2 = SCALE * y2 / jnp.maximum(nrm, 1e-12)
    y1a_ref[...] = y1[:, :H]
    y1b_ref[...] = y1[:, H:]
    y2a_ref[...] = y2[:, :H]
    y2b_ref[...] = y2[:, H:]


def _tc_scale_body(degp_ref, ya_ref, yb_ref, ua_ref, ub_ref):
    dinv = _dinv_block(degp_ref)
    ua_ref[...] = dinv * ya_ref[...]
    ub_ref[...] = dinv * yb_ref[...]


def _tc_fin_body(degp_ref, ua_ref, ub_ref, aa_ref, ab_ref, o_ref):
    dinv = _dinv_block(degp_ref)
    o_ref[...] = dinv * jnp.concatenate(
        [aa_ref[...] + ua_ref[...], ab_ref[...] + ub_ref[...]], axis=1)


RB = 400
TC_GRID = N // RB


def _row_spec(w):
    return pl.BlockSpec((RB, w), lambda i: (i, 0))


def _full_spec(h, w):
    return pl.BlockSpec((h, w), lambda i: (0, 0))


_degp_spec = pl.BlockSpec((2, RB, H), lambda i: (0, i, 0))

_tc_mats = pl.pallas_call(
    _tc_mats_body,
    grid=(TC_GRID,),
    in_specs=[_row_spec(D), _full_spec(D, D), _full_spec(1, D),
              _full_spec(D, D), _full_spec(1, D)],
    out_specs=[_row_spec(H)] * 4,
    out_shape=[jax.ShapeDtypeStruct((NPAD, H), jnp.float32)] * 4,
)

_tc_scale = pl.pallas_call(
    _tc_scale_body,
    grid=(TC_GRID,),
    in_specs=[_degp_spec] + [_row_spec(H)] * 2,
    out_specs=[_row_spec(H)] * 2,
    out_shape=[jax.ShapeDtypeStruct((NPAD, H), jnp.float32)] * 2,
)

_tc_fin = pl.pallas_call(
    _tc_fin_body,
    grid=(TC_GRID,),
    in_specs=[_degp_spec] + [_row_spec(H)] * 4,
    out_specs=_row_spec(D),
    out_shape=jax.ShapeDtypeStruct((N, D), jnp.float32),
)


def kernel(x, edge_index, W1, b1, W2, b2):
    src = edge_index[0].astype(jnp.int32)
    dst = edge_index[1].astype(jnp.int32)
    pad = N + (jnp.arange(EPAD - E, dtype=jnp.int32) % (NPAD - N))
    src = jnp.concatenate([src, pad]).reshape(NS, EP_W, W)
    dst = jnp.concatenate([dst, pad]).reshape(NS, EP_W, W)
    dst_deg = dst.reshape(NC * NS, ED_W, W)

    ones = jnp.ones((W, H), jnp.float32)
    z128 = jnp.zeros((NPAD, H), jnp.float32)

    degp = _sc_deg(dst_deg, ones, z128).reshape(2, NPAD, H)
    y1a, y1b, y2a, y2b = _tc_mats(x, W1, b1.reshape(1, D),
                                  W2, b2.reshape(1, D))
    u1a, u1b = _tc_scale(degp, y1a, y1b)
    a1a, a1b = _sc_prop(src, dst, u1a, u1b, z128)
    u2a, u2b = _tc_scale(degp, y2a, y2b)
    a2a, a2b = _sc_prop(src, dst, u2a, u2b, z128)
    x_ = _tc_fin(degp, u1a, u1b, a1a, a1b)
    h = _tc_fin(degp, u2a, u2b, a2a, a2b)
    return (h, x_)


# speedup vs baseline: 1.0695x; 1.0586x over previous
import dataclasses
import functools

import jax
import jax.numpy as jnp
from jax import lax
from jax.experimental import pallas as pl
from jax.experimental.pallas import tpu as pltpu
from jax.experimental.pallas import tpu_sc as plsc

N = 10000
D = 256
H = 128
E = 160000
SCALE = 1.8

NC = 2
NS = 16
W = 128
CHW = 40

NPAD = 10496
STRIPE = NPAD // NS

EPAD = 163840
ED_W = EPAD // (NC * NS * W)
EP_W = EPAD // (NS * W)

_mesh = plsc.VectorSubcoreMesh(core_axis_name="c", subcore_axis_name="s")


_deg2_cp = pltpu.CompilerParams()
if "needs_layout_passes" in pltpu.CompilerParams.__dataclass_fields__:
    _deg2_cp = dataclasses.replace(_deg2_cp, needs_layout_passes=False)


@functools.partial(
    pl.kernel,
    out_type=jax.ShapeDtypeStruct((NC * NS * NPAD,), jnp.float32),
    mesh=_mesh,
    compiler_params=_deg2_cp,
    scratch_types=[
        pltpu.VMEM((ED_W * W,), jnp.int32),
        pltpu.VMEM((NPAD,), jnp.float32),
    ],
)
def _sc_deg2(dst_hbm, z1_hbm, out_hbm, idx_v, hist_v):
    c = lax.axis_index("c")
    s = lax.axis_index("s")
    w = c * NS + s
    pltpu.sync_copy(dst_hbm.at[pl.ds(w * (ED_W * W), ED_W * W)], idx_v)
    pltpu.sync_copy(z1_hbm, hist_v)
    one = jnp.ones((16,), jnp.float32)

    @pl.loop(0, ED_W * W // 16)
    def _(i):
        plsc.addupdate_scatter(hist_v, [idx_v[pl.ds(i * 16, 16)]], one)

    pltpu.sync_copy(hist_v, out_hbm.at[pl.ds(w * NPAD, NPAD)])


@functools.partial(
    pl.kernel,
    out_type=[jax.ShapeDtypeStruct((NPAD, H), jnp.float32)] * 2,
    mesh=_mesh,
    scratch_types=[
        pltpu.VMEM((CHW, W), jnp.int32),
        pltpu.VMEM((CHW, W), jnp.int32),
        pltpu.VMEM((W, H), jnp.float32),
        pltpu.VMEM((W, H), jnp.float32),
        pltpu.VMEM_SHARED((NPAD, H), jnp.float32),
        pltpu.SemaphoreType.DMA,
        pltpu.SemaphoreType.DMA,
        pltpu.SemaphoreType.DMA,
        pltpu.SemaphoreType.DMA,
    ],
)
def _sc_prop(src_hbm, dst_hbm, t0, t1, z_hbm, o0, o1,
             idxs_v, idxd_v, upd_a, upd_b, acc_s,
             sem_ga, sem_gb, sem_sa, sem_sb):
    c = lax.axis_index("c")
    s = lax.axis_index("s")

    def one_pass(table, out):
        pltpu.sync_copy(z_hbm.at[pl.ds(s * STRIPE, STRIPE)],
                        acc_s.at[pl.ds(s * STRIPE, STRIPE)])
        plsc.subcore_barrier()

        for ch in range(EP_W // CHW):
            pltpu.sync_copy(src_hbm.at[s, pl.ds(ch * CHW, CHW)], idxs_v)
            pltpu.sync_copy(dst_hbm.at[s, pl.ds(ch * CHW, CHW)], idxd_v)

            pltpu.async_copy(table.at[idxs_v.at[0]], upd_a, sem_ga)

            @pl.loop(0, CHW // 2)
            def _(j):
                k = 2 * j
                pltpu.make_async_copy(
                    table.at[idxs_v.at[k]], upd_a, sem_ga).wait()
                pltpu.async_copy(upd_a, acc_s.at[idxd_v.at[k]], sem_sa,
                                 add=True)

                @pl.when(j > 0)
                def _():
                    pltpu.make_async_copy(
                        upd_b, acc_s.at[idxd_v.at[k - 1]], sem_sb).wait()

                pltpu.async_copy(table.at[idxs_v.at[k + 1]], upd_b, sem_gb)
                pltpu.make_async_copy(
                    table.at[idxs_v.at[k + 1]], upd_b, sem_gb).wait()
                pltpu.async_copy(upd_b, acc_s.at[idxd_v.at[k + 1]], sem_sb,
                                 add=True)
                pltpu.make_async_copy(
                    upd_a, acc_s.at[idxd_v.at[k]], sem_sa).wait()

                @pl.when(k + 2 < CHW)
                def _():
                    pltpu.async_copy(
                        table.at[idxs_v.at[k + 2]], upd_a, sem_ga)

            pltpu.make_async_copy(
                upd_b, acc_s.at[idxd_v.at[CHW - 1]], sem_sb).wait()

        plsc.subcore_barrier()
        pltpu.sync_copy(acc_s.at[pl.ds(s * STRIPE, STRIPE)],
                        out.at[pl.ds(s * STRIPE, STRIPE)])
        plsc.subcore_barrier()

    @pl.when(c == 0)
    def _():
        one_pass(t0, o0)

    @pl.when(c == 1)
    def _():
        one_pass(t1, o1)


def _dinv_block(degp_ref):
    deg = 1.0 + jnp.sum(degp_ref[...], axis=1, keepdims=True)
    return lax.rsqrt(deg)


def _tc_mats_body(x_ref, w1_ref, b1_ref, w2_ref, b2_ref,
                  y1a_ref, y1b_ref, y2a_ref, y2b_ref):
    dn = (((1,), (1,)), ((), ()))
    y1 = lax.dot_general(x_ref[...], w1_ref[...], dn,
                         preferred_element_type=jnp.float32) + b1_ref[...]
    y2 = lax.dot_general(x_ref[...], w2_ref[...], dn,
                         preferred_element_type=jnp.float32) + b2_ref[...]
    nrm = jnp.sqrt(jnp.sum(y2 * y2, axis=1, keepdims=True))
    y2 = SCALE * y2 / jnp.maximum(nrm, 1e-12)
    y1a_ref[...] = y1[:, :H]
    y1b_ref[...] = y1[:, H:]
    y2a_ref[...] = y2[:, :H]
    y2b_ref[...] = y2[:, H:]


def _tc_scale_body(degp_ref, ya_ref, yb_ref, ua_ref, ub_ref):
    dinv = _dinv_block(degp_ref)
    ua_ref[...] = dinv * ya_ref[...]
    ub_ref[...] = dinv * yb_ref[...]


def _tc_fin_body(degp_ref, ua_ref, ub_ref, aa_ref, ab_ref, o_ref):
    dinv = _dinv_block(degp_ref)
    o_ref[...] = dinv * jnp.concatenate(
        [aa_ref[...] + ua_ref[...], ab_ref[...] + ub_ref[...]], axis=1)


RB = 400
TC_GRID = N // RB


def _row_spec(w):
    return pl.BlockSpec((RB, w), lambda i: (i, 0))


def _full_spec(h, w):
    return pl.BlockSpec((h, w), lambda i: (0, 0))


_degp_spec = pl.BlockSpec((RB, NC * NS), lambda i: (i, 0))

_tc_mats = pl.pallas_call(
    _tc_mats_body,
    grid=(TC_GRID,),
    in_specs=[_row_spec(D), _full_spec(D, D), _full_spec(1, D),
              _full_spec(D, D), _full_spec(1, D)],
    out_specs=[_row_spec(H)] * 4,
    out_shape=[jax.ShapeDtypeStruct((NPAD, H), jnp.float32)] * 4,
)

_tc_scale = pl.pallas_call(
    _tc_scale_body,
    grid=(TC_GRID,),
    in_specs=[_degp_spec] + [_row_spec(H)] * 2,
    out_specs=[_row_spec(H)] * 2,
    out_shape=[jax.ShapeDtypeStruct((NPAD, H), jnp.float32)] * 2,
)

_tc_fin = pl.pallas_call(
    _tc_fin_body,
    grid=(TC_GRID,),
    in_specs=[_degp_spec] + [_row_spec(H)] * 4,
    out_specs=_row_spec(D),
    out_shape=jax.ShapeDtypeStruct((N, D), jnp.float32),
)


def kernel(x, edge_index, W1, b1, W2, b2):
    src = edge_index[0].astype(jnp.int32)
    dst = edge_index[1].astype(jnp.int32)
    pad = N + (jnp.arange(EPAD - E, dtype=jnp.int32) % (NPAD - N))
    src = jnp.concatenate([src, pad]).reshape(NS, EP_W, W)
    dst = jnp.concatenate([dst, pad]).reshape(NS, EP_W, W)

    z128 = jnp.zeros((NPAD, H), jnp.float32)
    z1 = jnp.zeros((NPAD,), jnp.float32)

    degp = _sc_deg2(dst.reshape(EPAD), z1).reshape(NC * NS, NPAD).T
    y1a, y1b, y2a, y2b = _tc_mats(x, W1, b1.reshape(1, D),
                                  W2, b2.reshape(1, D))
    u1a, u1b = _tc_scale(degp, y1a, y1b)
    a1a, a1b = _sc_prop(src, dst, u1a, u1b, z128)
    u2a, u2b = _tc_scale(degp, y2a, y2b)
    a2a, a2b = _sc_prop(src, dst, u2a, u2b, z128)
    x_ = _tc_fin(degp, u1a, u1b, a1a, a1b)
    h = _tc_fin(degp, u2a, u2b, a2a, a2b)
    return (h, x_)


# speedup vs baseline: 1.1112x; 1.0390x over previous
import dataclasses
import functools

import jax
import jax.numpy as jnp
from jax import lax
from jax.experimental import pallas as pl
from jax.experimental.pallas import tpu as pltpu
from jax.experimental.pallas import tpu_sc as plsc

N = 10000
D = 256
H = 128
E = 160000
SCALE = 1.8

NC = 2
NS = 16
W = 128
CHW = 40

NPAD = 10496
STRIPE = NPAD // NS

EPAD = 163840
ED_W = EPAD // (NC * NS * W)
EP_W = EPAD // (NS * W)

_mesh = plsc.VectorSubcoreMesh(core_axis_name="c", subcore_axis_name="s")


_deg2_cp = pltpu.CompilerParams()
if "needs_layout_passes" in pltpu.CompilerParams.__dataclass_fields__:
    _deg2_cp = dataclasses.replace(_deg2_cp, needs_layout_passes=False)


@functools.partial(
    pl.kernel,
    out_type=jax.ShapeDtypeStruct((NC * NS * NPAD,), jnp.float32),
    mesh=_mesh,
    compiler_params=_deg2_cp,
    scratch_types=[
        pltpu.VMEM((ED_W * W,), jnp.int32),
        pltpu.VMEM((NPAD,), jnp.float32),
    ],
)
def _sc_deg2(dst_hbm, z1_hbm, out_hbm, idx_v, hist_v):
    c = lax.axis_index("c")
    s = lax.axis_index("s")
    w = c * NS + s
    pltpu.sync_copy(dst_hbm.at[pl.ds(w * (ED_W * W), ED_W * W)], idx_v)
    pltpu.sync_copy(z1_hbm, hist_v)
    one = jnp.ones((16,), jnp.float32)

    @pl.loop(0, ED_W * W // 16)
    def _(i):
        plsc.addupdate_scatter(hist_v, [idx_v[pl.ds(i * 16, 16)]], one)

    pltpu.sync_copy(hist_v, out_hbm.at[pl.ds(w * NPAD, NPAD)])


@functools.partial(
    pl.kernel,
    out_type=[jax.ShapeDtypeStruct((NPAD, H), jnp.float32)] * 2,
    mesh=_mesh,
    scratch_types=[
        pltpu.VMEM((CHW, W), jnp.int32),
        pltpu.VMEM((CHW, W), jnp.int32),
        pltpu.VMEM((W, H), jnp.float32),
        pltpu.VMEM((W, H), jnp.float32),
        pltpu.VMEM_SHARED((NPAD, H), jnp.float32),
        pltpu.SemaphoreType.DMA,
        pltpu.SemaphoreType.DMA,
        pltpu.SemaphoreType.DMA,
        pltpu.SemaphoreType.DMA,
    ],
)
def _sc_prop(src_hbm, dst_hbm, t0, t1, z_hbm, o0, o1,
             idxs_v, idxd_v, upd_a, upd_b, acc_s,
             sem_ga, sem_gb, sem_sa, sem_sb):
    c = lax.axis_index("c")
    s = lax.axis_index("s")

    def one_pass(table, out):
        pltpu.sync_copy(z_hbm.at[pl.ds(s * STRIPE, STRIPE)],
                        acc_s.at[pl.ds(s * STRIPE, STRIPE)])
        plsc.subcore_barrier()

        for ch in range(EP_W // CHW):
            pltpu.sync_copy(src_hbm.at[s, pl.ds(ch * CHW, CHW)], idxs_v)
            pltpu.sync_copy(dst_hbm.at[s, pl.ds(ch * CHW, CHW)], idxd_v)

            pltpu.async_copy(table.at[idxs_v.at[0]], upd_a, sem_ga)

            @pl.loop(0, CHW // 2)
            def _(j):
                k = 2 * j
                pltpu.make_async_copy(
                    table.at[idxs_v.at[k]], upd_a, sem_ga).wait()
                pltpu.async_copy(upd_a, acc_s.at[idxd_v.at[k]], sem_sa,
                                 add=True)

                @pl.when(j > 0)
                def _():
                    pltpu.make_async_copy(
                        upd_b, acc_s.at[idxd_v.at[k - 1]], sem_sb).wait()

                pltpu.async_copy(table.at[idxs_v.at[k + 1]], upd_b, sem_gb)
                pltpu.make_async_copy(
                    table.at[idxs_v.at[k + 1]], upd_b, sem_gb).wait()
                pltpu.async_copy(upd_b, acc_s.at[idxd_v.at[k + 1]], sem_sb,
                                 add=True)
                pltpu.make_async_copy(
                    upd_a, acc_s.at[idxd_v.at[k]], sem_sa).wait()

                @pl.when(k + 2 < CHW)
                def _():
                    pltpu.async_copy(
                        table.at[idxs_v.at[k + 2]], upd_a, sem_ga)

            pltpu.make_async_copy(
                upd_b, acc_s.at[idxd_v.at[CHW - 1]], sem_sb).wait()

        plsc.subcore_barrier()
        pltpu.sync_copy(acc_s.at[pl.ds(s * STRIPE, STRIPE)],
                        out.at[pl.ds(s * STRIPE, STRIPE)])
        plsc.subcore_barrier()

    @pl.when(c == 0)
    def _():
        one_pass(t0, o0)

    @pl.when(c == 1)
    def _():
        one_pass(t1, o1)


def _dinv_block(degp_ref):
    deg = 1.0 + jnp.sum(degp_ref[...], axis=1, keepdims=True)
    return lax.rsqrt(deg)


def _tc_mats_body(x_ref, w1_ref, b1_ref, w2_ref, b2_ref, degp_ref,
                  u1a_ref, u1b_ref, u2a_ref, u2b_ref):
    dinv = _dinv_block(degp_ref)
    dn = (((1,), (1,)), ((), ()))
    y1 = lax.dot_general(x_ref[...], w1_ref[...], dn,
                         preferred_element_type=jnp.float32) + b1_ref[...]
    u1 = dinv * y1
    y2 = lax.dot_general(x_ref[...], w2_ref[...], dn,
                         preferred_element_type=jnp.float32) + b2_ref[...]
    nrm = jnp.sqrt(jnp.sum(y2 * y2, axis=1, keepdims=True))
    u2 = dinv * (SCALE / jnp.maximum(nrm, 1e-12)) * y2
    u1a_ref[...] = u1[:, :H]
    u1b_ref[...] = u1[:, H:]
    u2a_ref[...] = u2[:, :H]
    u2b_ref[...] = u2[:, H:]


def _tc_fin_body(degp_ref, ua_ref, ub_ref, aa_ref, ab_ref, o_ref):
    dinv = _dinv_block(degp_ref)
    o_ref[...] = dinv * jnp.concatenate(
        [aa_ref[...] + ua_ref[...], ab_ref[...] + ub_ref[...]], axis=1)


RB = 400
TC_GRID = N // RB


def _row_spec(w):
    return pl.BlockSpec((RB, w), lambda i: (i, 0))


def _full_spec(h, w):
    return pl.BlockSpec((h, w), lambda i: (0, 0))


_degp_spec = pl.BlockSpec((RB, NC * NS), lambda i: (i, 0))

_tc_mats = pl.pallas_call(
    _tc_mats_body,
    grid=(TC_GRID,),
    in_specs=[_row_spec(D), _full_spec(D, D), _full_spec(1, D),
              _full_spec(D, D), _full_spec(1, D), _degp_spec],
    out_specs=[_row_spec(H)] * 4,
    out_shape=[jax.ShapeDtypeStruct((NPAD, H), jnp.float32)] * 4,
)

_tc_fin = pl.pallas_call(
    _tc_fin_body,
    grid=(TC_GRID,),
    in_specs=[_degp_spec] + [_row_spec(H)] * 4,
    out_specs=_row_spec(D),
    out_shape=jax.ShapeDtypeStruct((N, D), jnp.float32),
)


def kernel(x, edge_index, W1, b1, W2, b2):
    src = edge_index[0].astype(jnp.int32)
    dst = edge_index[1].astype(jnp.int32)
    pad = N + (jnp.arange(EPAD - E, dtype=jnp.int32) % (NPAD - N))
    src = jnp.concatenate([src, pad]).reshape(NS, EP_W, W)
    dst = jnp.concatenate([dst, pad]).reshape(NS, EP_W, W)

    z128 = jnp.zeros((NPAD, H), jnp.float32)
    z1 = jnp.zeros((NPAD,), jnp.float32)

    degp = _sc_deg2(dst.reshape(EPAD), z1).reshape(NC * NS, NPAD).T
    u1a, u1b, u2a, u2b = _tc_mats(x, W1, b1.reshape(1, D),
                                  W2, b2.reshape(1, D), degp)
    a1a, a1b = _sc_prop(src, dst, u1a, u1b, z128)
    a2a, a2b = _sc_prop(src, dst, u2a, u2b, z128)
    x_ = _tc_fin(degp, u1a, u1b, a1a, a1b)
    h = _tc_fin(degp, u2a, u2b, a2a, a2b)
    return (h, x_)
